# baseline (speedup 1.0000x reference)
import functools

import jax
import jax.numpy as jnp
from jax import lax
from jax.experimental import pallas as pl
from jax.experimental.pallas import tpu as pltpu
from jax.experimental.pallas import tpu_sc as plsc

_C = 256
_OUT = 7
_BINS = _OUT * _OUT
_PAIRS = 16
_ROWS_PER_ROI = _BINS * _PAIRS
_CHUNK = 112
_NCHUNK = 7
_R = 512
_NW = 32
_RPW = _R // _NW

_LEVEL_W = (200.0, 100.0, 50.0, 25.0)
_LEVEL_SCALE = (0.25, 0.125, 0.0625, 0.03125)
_LEVEL_OFF = (0.0, 80000.0, 100000.0, 105000.0)


def _sample_idx_w(bin_i, pair_i, x1s, y1s, bw2, bh2, wf, wi, base):
    f32 = jnp.float32
    binf = bin_i.astype(f32)
    phf = jnp.floor(binf / 7.0)
    pwf = binf - 7.0 * phf
    iy = (pair_i >> 1) & 1
    ix = pair_i & 1
    cc = pair_i >> 2
    cy = cc >> 1
    cx = cc & 1
    syf = 2.0 * phf + iy.astype(f32)
    sxf = 2.0 * pwf + ix.astype(f32)
    yf = y1s + (syf + 0.5) * bh2
    xf = x1s + (sxf + 0.5) * bw2
    valid = ((yf >= -1.0) & (yf <= wf)) & ((xf >= -1.0) & (xf <= wf))
    ycl = jnp.clip(yf, 0.0, wf - 1.0)
    xcl = jnp.clip(xf, 0.0, wf - 1.0)
    y0f = jnp.floor(ycl)
    x0f = jnp.floor(xcl)
    ly = ycl - y0f
    lx = xcl - x0f
    y0 = y0f.astype(jnp.int32)
    x0 = x0f.astype(jnp.int32)
    y1i = jnp.minimum(y0 + 1, wi - 1)
    x1i = jnp.minimum(x0 + 1, wi - 1)
    wy = jnp.where(cy == 0, 1.0 - ly, ly)
    wx = jnp.where(cx == 0, 1.0 - lx, lx)
    yc = jnp.where(cy == 0, y0, y1i)
    xc = jnp.where(cx == 0, x0, x1i)
    weight = wy * wx * (0.25 * valid.astype(f32))
    idx = base + yc * wi + xc
    return idx, weight


def _coef_body(boxes_ref, bids_ref, idx_ref, w_ref):
    f32 = jnp.float32
    boxes = boxes_ref[...]
    b = bids_ref[...].astype(jnp.int32)
    x1 = boxes[:, 0:1]
    y1 = boxes[:, 1:2]
    x2 = boxes[:, 2:3]
    y2 = boxes[:, 3:4]
    area = (x2 - x1) * (y2 - y1)
    s = jnp.sqrt(jnp.maximum(area, 0.0))
    t = jnp.floor(4.0 + jnp.log2(s / 224.0 + 1e-6))
    lvl = jnp.clip(t, 2.0, 5.0) - 2.0

    def per_level(vals):
        v0, v1, v2, v3 = (jnp.asarray(v, f32) for v in vals)
        return jnp.where(lvl < 0.5, v0,
                         jnp.where(lvl < 1.5, v1,
                                   jnp.where(lvl < 2.5, v2, v3)))

    scale = per_level(_LEVEL_SCALE)
    wf = per_level(_LEVEL_W)
    off = per_level(_LEVEL_OFF)
    wi = wf.astype(jnp.int32)
    base = off.astype(jnp.int32) + b * (wi * wi)
    x1s = x1 * scale
    y1s = y1 * scale
    x2s = x2 * scale
    y2s = y2 * scale
    roi_w = jnp.maximum(x2s - x1s, 1.0)
    roi_h = jnp.maximum(y2s - y1s, 1.0)
    bw2 = (roi_w / 7.0) * 0.5
    bh2 = (roi_h / 7.0) * 0.5

    nb = boxes.shape[0]
    jb = lax.broadcasted_iota(jnp.int32, (nb, _ROWS_PER_ROI), 1)
    idx, w = _sample_idx_w(jb >> 4, jb & 15, x1s, y1s, bw2, bh2, wf, wi, base)
    idx_ref[...] = idx
    w_ref[...] = w


def _coefs(boxes, bids):
    blk = 64
    grid = _R // blk
    return pl.pallas_call(
        _coef_body,
        grid=(grid,),
        in_specs=[
            pl.BlockSpec((blk, 4), lambda i: (i, 0)),
            pl.BlockSpec((blk, 1), lambda i: (i, 0)),
        ],
        out_specs=[
            pl.BlockSpec((blk, _ROWS_PER_ROI), lambda i: (i, 0)),
            pl.BlockSpec((blk, _ROWS_PER_ROI), lambda i: (i, 0)),
        ],
        out_shape=[
            jax.ShapeDtypeStruct((_R, _ROWS_PER_ROI), jnp.int32),
            jax.ShapeDtypeStruct((_R, _ROWS_PER_ROI), jnp.float32),
        ],
    )(boxes, bids)


def _pack_body(t_ref, o_ref):
    x = t_ref[...]
    lo = lax.bitcast_convert_type(
        x[:, :128].astype(jnp.bfloat16), jnp.uint16).astype(jnp.int32)
    hi = lax.bitcast_convert_type(
        x[:, 128:].astype(jnp.bfloat16), jnp.uint16).astype(jnp.int32)
    o_ref[...] = (hi << 16) | lo


def _pack_table(table):
    rows = table.shape[0]
    blk = 1024
    return pl.pallas_call(
        _pack_body,
        grid=(pl.cdiv(rows, blk),),
        in_specs=[pl.BlockSpec((blk, _C), lambda i: (i, 0))],
        out_specs=pl.BlockSpec((blk, _C // 2), lambda i: (i, 0)),
        out_shape=jax.ShapeDtypeStruct((rows, _C // 2), jnp.int32),
    )(table)


def _lane_broadcast(vec, j):
    idx = jnp.full((16, 1), j, jnp.int32)
    return lax.gather(
        vec, idx,
        dimension_numbers=lax.GatherDimensionNumbers(
            offset_dims=(), collapsed_slice_dims=(0,), start_index_map=(0,)),
        slice_sizes=(1,),
        mode=lax.GatherScatterMode.PROMISE_IN_BOUNDS)


def _sc_pool(table, idx4, w3):
    mesh = plsc.VectorSubcoreMesh(core_axis_name="c", subcore_axis_name="s")

    @functools.partial(
        pl.kernel,
        mesh=mesh,
        out_type=jax.ShapeDtypeStruct((_R, _NCHUNK * 8, _C), jnp.float32),
        scratch_types=[
            pltpu.VMEM((_RPW, _NCHUNK, _CHUNK), jnp.int32),
            pltpu.VMEM((_RPW, _ROWS_PER_ROI), jnp.float32),
            pltpu.VMEM((_CHUNK, _C // 2), jnp.int32),
            pltpu.VMEM((_CHUNK, _C // 2), jnp.int32),
            pltpu.VMEM((_CHUNK, _C // 2), jnp.int32),
            pltpu.VMEM((8, _C), jnp.float32),
            pltpu.VMEM((8, _C), jnp.float32),
            pltpu.SemaphoreType.DMA,
            pltpu.SemaphoreType.DMA,
            pltpu.SemaphoreType.DMA,
            pltpu.SemaphoreType.DMA,
            pltpu.SemaphoreType.DMA,
        ],
    )
    def k(table_hbm, idx_hbm, w_hbm, out_hbm, idx_v, w_v, rows0, rows_a,
          rows_b, acc_a, acc_b, sem0, sem_a, sem_b, sem_wa, sem_wb):
        wid = lax.axis_index("s") * 2 + lax.axis_index("c")
        pltpu.sync_copy(idx_hbm.at[wid], idx_v)
        pltpu.sync_copy(w_hbm.at[wid], w_v)
        bufs = (rows0, rows_a, rows_b, rows_a, rows_b, rows_a, rows_b)
        sems = (sem0, sem_a, sem_b, sem_a, sem_b, sem_a, sem_b)
        accs_v = (acc_a, acc_b)
        wsems = (sem_wa, sem_wb)
        pltpu.async_copy(table_hbm.at[idx_v.at[0, 0]], rows0, sem0)

        def roi_iter(i, carry):
            r = wid * _RPW + i
            pltpu.make_async_copy(table_hbm.at[idx_v.at[i, 0]], rows0,
                                  sem0).wait()
            cps = {}
            for c in range(_NCHUNK):
                if c > 0:
                    cps[c].wait()
                if c + 1 < _NCHUNK:
                    cps[c + 1] = pltpu.async_copy(
                        table_hbm.at[idx_v.at[i, c + 1]], bufs[c + 1],
                        sems[c + 1])
                else:
                    @pl.when(i + 1 < _RPW)
                    def _():
                        pltpu.async_copy(table_hbm.at[idx_v.at[i + 1, 0]],
                                         rows0, sem0)
                buf = bufs[c]
                acc_buf = accs_v[c % 2]
                wsem = wsems[c % 2]
                wbase = c * _CHUNK
                if c >= 2:
                    pltpu.make_async_copy(
                        acc_buf, out_hbm.at[r, pl.ds(0, 8)], wsem).wait()
                else:
                    @pl.when(i > 0)
                    def _():
                        pltpu.make_async_copy(
                            acc_buf, out_hbm.at[r, pl.ds(0, 8)],
                            wsem).wait()

                def bin_iter(pw, _, c=c, buf=buf, acc_buf=acc_buf,
                             wbase=wbase):
                    accs = [jnp.zeros((16,), jnp.float32) for _ in range(16)]
                    wv = w_v[i, pl.ds(wbase + pw * _PAIRS, _PAIRS)]
                    hi_mask = jnp.full((16,), -65536, jnp.int32)
                    for j in range(_PAIRS):
                        wj = _lane_broadcast(wv, j)
                        row = pw * _PAIRS + j
                        for g in range(8):
                            u = buf[row, pl.ds(16 * g, 16)]
                            ulo = lax.bitcast_convert_type(
                                u << 16, jnp.float32)
                            uhi = lax.bitcast_convert_type(
                                u & hi_mask, jnp.float32)
                            accs[g] = accs[g] + wj * ulo
                            accs[8 + g] = accs[8 + g] + wj * uhi
                    for v in range(16):
                        acc_buf[pw, pl.ds(16 * v, 16)] = accs[v]
                    return _

                lax.fori_loop(0, _OUT, bin_iter, 0)
                pltpu.async_copy(acc_buf, out_hbm.at[r, pl.ds(c * 8, 8)],
                                 wsem)
            return carry

        lax.fori_loop(0, _RPW, roi_iter, 0)
        pltpu.make_async_copy(acc_b, out_hbm.at[0, pl.ds(0, 8)],
                              sem_wb).wait()
        pltpu.make_async_copy(acc_a, out_hbm.at[0, pl.ds(0, 8)],
                              sem_wa).wait()

    return k(table, idx4, w3)


def kernel(feat0, feat1, feat2, feat3, boxes, batch_ids):
    table = jnp.concatenate(
        [f.transpose(0, 2, 3, 1).reshape(-1, _C)
         for f in (feat0, feat1, feat2, feat3)], axis=0)
    table = _pack_table(table)
    bids = batch_ids.astype(jnp.int32).reshape(_R, 1)
    idx2, w2 = _coefs(boxes.astype(jnp.float32), bids)
    idx4 = idx2.reshape(_NW, _RPW, _NCHUNK, _CHUNK)
    w3 = w2.reshape(_NW, _RPW, _ROWS_PER_ROI)
    padded = _sc_pool(table, idx4, w3)
    pooled = padded.reshape(_R, _NCHUNK, 8, _C)[:, :, :_OUT]
    pooled = pooled.reshape(_R, _BINS, _C)
    return pooled.transpose(0, 2, 1).reshape(_R, _C, _OUT, _OUT)


# speedup vs baseline: 1.1664x; 1.1664x over previous
import functools

import jax
import jax.numpy as jnp
from jax import lax
from jax.experimental import pallas as pl
from jax.experimental.pallas import tpu as pltpu
from jax.experimental.pallas import tpu_sc as plsc

_C = 256
_OUT = 7
_BINS = _OUT * _OUT
_PAIRS = 16
_ROWS_PER_ROI = _BINS * _PAIRS
_CHUNK = 112
_NCHUNK = 7
_R = 512
_NW = 32
_RPW = _R // _NW

_LEVEL_W = (200.0, 100.0, 50.0, 25.0)
_LEVEL_SCALE = (0.25, 0.125, 0.0625, 0.03125)
_LEVEL_OFF = (0.0, 80000.0, 100000.0, 105000.0)


def _sample_idx_w(bin_i, pair_i, x1s, y1s, bw2, bh2, wf, wi, base):
    f32 = jnp.float32
    binf = bin_i.astype(f32)
    phf = jnp.floor(binf / 7.0)
    pwf = binf - 7.0 * phf
    iy = (pair_i >> 1) & 1
    ix = pair_i & 1
    cc = pair_i >> 2
    cy = cc >> 1
    cx = cc & 1
    syf = 2.0 * phf + iy.astype(f32)
    sxf = 2.0 * pwf + ix.astype(f32)
    yf = y1s + (syf + 0.5) * bh2
    xf = x1s + (sxf + 0.5) * bw2
    valid = ((yf >= -1.0) & (yf <= wf)) & ((xf >= -1.0) & (xf <= wf))
    ycl = jnp.clip(yf, 0.0, wf - 1.0)
    xcl = jnp.clip(xf, 0.0, wf - 1.0)
    y0f = jnp.floor(ycl)
    x0f = jnp.floor(xcl)
    ly = ycl - y0f
    lx = xcl - x0f
    y0 = y0f.astype(jnp.int32)
    x0 = x0f.astype(jnp.int32)
    y1i = jnp.minimum(y0 + 1, wi - 1)
    x1i = jnp.minimum(x0 + 1, wi - 1)
    wy = jnp.where(cy == 0, 1.0 - ly, ly)
    wx = jnp.where(cx == 0, 1.0 - lx, lx)
    yc = jnp.where(cy == 0, y0, y1i)
    xc = jnp.where(cx == 0, x0, x1i)
    weight = wy * wx * (0.25 * valid.astype(f32))
    idx = base + yc * wi + xc
    return idx, weight


def _coef_body(boxes_ref, bids_ref, idx_ref, w_ref):
    f32 = jnp.float32
    boxes = boxes_ref[...]
    b = bids_ref[...].astype(jnp.int32)
    x1 = boxes[:, 0:1]
    y1 = boxes[:, 1:2]
    x2 = boxes[:, 2:3]
    y2 = boxes[:, 3:4]
    area = (x2 - x1) * (y2 - y1)
    s = jnp.sqrt(jnp.maximum(area, 0.0))
    t = jnp.floor(4.0 + jnp.log2(s / 224.0 + 1e-6))
    lvl = jnp.clip(t, 2.0, 5.0) - 2.0

    def per_level(vals):
        v0, v1, v2, v3 = (jnp.asarray(v, f32) for v in vals)
        return jnp.where(lvl < 0.5, v0,
                         jnp.where(lvl < 1.5, v1,
                                   jnp.where(lvl < 2.5, v2, v3)))

    scale = per_level(_LEVEL_SCALE)
    wf = per_level(_LEVEL_W)
    off = per_level(_LEVEL_OFF)
    wi = wf.astype(jnp.int32)
    base = off.astype(jnp.int32) + b * (wi * wi)
    x1s = x1 * scale
    y1s = y1 * scale
    x2s = x2 * scale
    y2s = y2 * scale
    roi_w = jnp.maximum(x2s - x1s, 1.0)
    roi_h = jnp.maximum(y2s - y1s, 1.0)
    bw2 = (roi_w / 7.0) * 0.5
    bh2 = (roi_h / 7.0) * 0.5

    nb = boxes.shape[0]
    jb = lax.broadcasted_iota(jnp.int32, (nb, _ROWS_PER_ROI), 1)
    idx, w = _sample_idx_w(jb >> 4, jb & 15, x1s, y1s, bw2, bh2, wf, wi, base)
    idx_ref[...] = idx
    w_ref[...] = w


def _coefs(boxes, bids):
    blk = 64
    grid = _R // blk
    return pl.pallas_call(
        _coef_body,
        grid=(grid,),
        in_specs=[
            pl.BlockSpec((blk, 4), lambda i: (i, 0)),
            pl.BlockSpec((blk, 1), lambda i: (i, 0)),
        ],
        out_specs=[
            pl.BlockSpec((blk, _ROWS_PER_ROI), lambda i: (i, 0)),
            pl.BlockSpec((blk, _ROWS_PER_ROI), lambda i: (i, 0)),
        ],
        out_shape=[
            jax.ShapeDtypeStruct((_R, _ROWS_PER_ROI), jnp.int32),
            jax.ShapeDtypeStruct((_R, _ROWS_PER_ROI), jnp.float32),
        ],
    )(boxes, bids)


def _pack_body(t_ref, o_ref):
    x = t_ref[...]
    lo = lax.bitcast_convert_type(
        x[:, :128].astype(jnp.bfloat16), jnp.uint16).astype(jnp.int32)
    hi = lax.bitcast_convert_type(
        x[:, 128:].astype(jnp.bfloat16), jnp.uint16).astype(jnp.int32)
    o_ref[...] = (hi << 16) | lo


def _pack_table(table):
    rows = table.shape[0]
    blk = 1024
    return pl.pallas_call(
        _pack_body,
        grid=(pl.cdiv(rows, blk),),
        in_specs=[pl.BlockSpec((blk, _C), lambda i: (i, 0))],
        out_specs=pl.BlockSpec((blk, _C // 2), lambda i: (i, 0)),
        out_shape=jax.ShapeDtypeStruct((rows, _C // 2), jnp.int32),
    )(table)


def _lane_broadcast(vec, j):
    idx = jnp.full((16, 1), j, jnp.int32)
    return lax.gather(
        vec, idx,
        dimension_numbers=lax.GatherDimensionNumbers(
            offset_dims=(), collapsed_slice_dims=(0,), start_index_map=(0,)),
        slice_sizes=(1,),
        mode=lax.GatherScatterMode.PROMISE_IN_BOUNDS)


def _sc_pool(table, idx4, w3):
    mesh = plsc.VectorSubcoreMesh(core_axis_name="c", subcore_axis_name="s")

    @functools.partial(
        pl.kernel,
        mesh=mesh,
        out_type=jax.ShapeDtypeStruct((_R, _NCHUNK * 8, _C), jnp.float32),
        scratch_types=[
            pltpu.VMEM((_RPW, _NCHUNK, _CHUNK), jnp.int32),
            pltpu.VMEM((_RPW, _ROWS_PER_ROI), jnp.float32),
            pltpu.VMEM((_CHUNK, _C // 2), jnp.int32),
            pltpu.VMEM((_CHUNK, _C // 2), jnp.int32),
            pltpu.VMEM((_CHUNK, _C // 2), jnp.int32),
            pltpu.VMEM((8, _C), jnp.float32),
            pltpu.VMEM((8, _C), jnp.float32),
            pltpu.SemaphoreType.DMA,
            pltpu.SemaphoreType.DMA,
            pltpu.SemaphoreType.DMA,
            pltpu.SemaphoreType.DMA,
            pltpu.SemaphoreType.DMA,
        ],
    )
    def k(table_hbm, idx_hbm, w_hbm, out_hbm, idx_v, w_v, rows0, rows_a,
          rows_b, acc_a, acc_b, sem0, sem_a, sem_b, sem_wa, sem_wb):
        wid = lax.axis_index("s") * 2 + lax.axis_index("c")
        pltpu.sync_copy(idx_hbm.at[wid], idx_v)
        pltpu.sync_copy(w_hbm.at[wid], w_v)
        bufs = (rows0, rows_a, rows_b, rows_a, rows_b, rows_a, rows_b)
        sems = (sem0, sem_a, sem_b, sem_a, sem_b, sem_a, sem_b)
        accs_v = (acc_a, acc_b)
        wsems = (sem_wa, sem_wb)
        pltpu.async_copy(table_hbm.at[idx_v.at[0, 0]], rows0, sem0)

        def roi_iter(i, carry):
            r = wid * _RPW + i
            pltpu.make_async_copy(table_hbm.at[idx_v.at[i, 0]], rows0,
                                  sem0).wait()
            cps = {}
            for c in range(_NCHUNK):
                if c > 0:
                    cps[c].wait()
                if c + 1 < _NCHUNK:
                    cps[c + 1] = pltpu.async_copy(
                        table_hbm.at[idx_v.at[i, c + 1]], bufs[c + 1],
                        sems[c + 1])
                else:
                    @pl.when(i + 1 < _RPW)
                    def _():
                        pltpu.async_copy(table_hbm.at[idx_v.at[i + 1, 0]],
                                         rows0, sem0)
                buf = bufs[c]
                acc_buf = accs_v[c % 2]
                wsem = wsems[c % 2]
                wbase = c * _CHUNK
                if c >= 2:
                    pltpu.make_async_copy(
                        acc_buf, out_hbm.at[r, pl.ds(0, 8)], wsem).wait()
                else:
                    @pl.when(i > 0)
                    def _():
                        pltpu.make_async_copy(
                            acc_buf, out_hbm.at[r, pl.ds(0, 8)],
                            wsem).wait()

                def bin_iter(pw, _, c=c, buf=buf, acc_buf=acc_buf,
                             wbase=wbase):
                    accs = [jnp.zeros((16,), jnp.float32) for _ in range(16)]
                    wv = w_v[i, pl.ds(wbase + pw * _PAIRS, _PAIRS)]
                    hi_mask = jnp.full((16,), -65536, jnp.int32)
                    for j in range(_PAIRS):
                        wj = _lane_broadcast(wv, j)
                        row = pw * _PAIRS + j
                        for g in range(8):
                            u = buf[row, pl.ds(16 * g, 16)]
                            ulo = lax.bitcast_convert_type(
                                u << 16, jnp.float32)
                            uhi = lax.bitcast_convert_type(
                                u & hi_mask, jnp.float32)
                            accs[g] = accs[g] + wj * ulo
                            accs[8 + g] = accs[8 + g] + wj * uhi
                    for v in range(16):
                        acc_buf[pw, pl.ds(16 * v, 16)] = accs[v]
                    return _

                lax.fori_loop(0, _OUT, bin_iter, 0)
                pltpu.async_copy(acc_buf, out_hbm.at[r, pl.ds(c * 8, 8)],
                                 wsem)
            return carry

        lax.fori_loop(0, _RPW, roi_iter, 0)
        pltpu.make_async_copy(acc_b, out_hbm.at[0, pl.ds(0, 8)],
                              sem_wb).wait()
        pltpu.make_async_copy(acc_a, out_hbm.at[0, pl.ds(0, 8)],
                              sem_wa).wait()

    return k(table, idx4, w3)


def kernel(feat0, feat1, feat2, feat3, boxes, batch_ids):
    def _packed(f):
        lo = lax.bitcast_convert_type(
            f[:, :128].astype(jnp.bfloat16), jnp.uint16).astype(jnp.int32)
        hi = lax.bitcast_convert_type(
            f[:, 128:].astype(jnp.bfloat16), jnp.uint16).astype(jnp.int32)
        return (hi << 16) | lo

    packed = [lax.optimization_barrier(_packed(f))
              for f in (feat0, feat1, feat2, feat3)]
    table = jnp.concatenate(
        [p.transpose(0, 2, 3, 1).reshape(-1, _C // 2) for p in packed],
        axis=0)
    bids = batch_ids.astype(jnp.int32).reshape(_R, 1)
    idx2, w2 = _coefs(boxes.astype(jnp.float32), bids)
    idx4 = idx2.reshape(_NW, _RPW, _NCHUNK, _CHUNK)
    w3 = w2.reshape(_NW, _RPW, _ROWS_PER_ROI)
    padded = _sc_pool(table, idx4, w3)
    pooled = padded.reshape(_R, _NCHUNK, 8, _C)[:, :, :_OUT]
    pooled = pooled.reshape(_R, _BINS, _C)
    return pooled.transpose(0, 2, 1).reshape(_R, _C, _OUT, _OUT)


# speedup vs baseline: 1.2626x; 1.0825x over previous
import functools

import jax
import jax.numpy as jnp
from jax import lax
from jax.experimental import pallas as pl
from jax.experimental.pallas import tpu as pltpu
from jax.experimental.pallas import tpu_sc as plsc

_C = 256
_OUT = 7
_BINS = _OUT * _OUT
_PAIRS = 16
_ROWS_PER_ROI = _BINS * _PAIRS
_CHUNK = 112
_NCHUNK = 7
_R = 512
_NW = 32
_RPW = _R // _NW

_LEVEL_W = (200.0, 100.0, 50.0, 25.0)
_LEVEL_SCALE = (0.25, 0.125, 0.0625, 0.03125)
_LEVEL_OFF = (0.0, 80000.0, 100000.0, 105000.0)


def _sample_idx_w(bin_i, pair_i, x1s, y1s, bw2, bh2, wf, wi, base):
    f32 = jnp.float32
    binf = bin_i.astype(f32)
    phf = jnp.floor(binf / 7.0)
    pwf = binf - 7.0 * phf
    iy = (pair_i >> 1) & 1
    ix = pair_i & 1
    cc = pair_i >> 2
    cy = cc >> 1
    cx = cc & 1
    syf = 2.0 * phf + iy.astype(f32)
    sxf = 2.0 * pwf + ix.astype(f32)
    yf = y1s + (syf + 0.5) * bh2
    xf = x1s + (sxf + 0.5) * bw2
    valid = ((yf >= -1.0) & (yf <= wf)) & ((xf >= -1.0) & (xf <= wf))
    ycl = jnp.clip(yf, 0.0, wf - 1.0)
    xcl = jnp.clip(xf, 0.0, wf - 1.0)
    y0f = jnp.floor(ycl)
    x0f = jnp.floor(xcl)
    ly = ycl - y0f
    lx = xcl - x0f
    y0 = y0f.astype(jnp.int32)
    x0 = x0f.astype(jnp.int32)
    y1i = jnp.minimum(y0 + 1, wi - 1)
    x1i = jnp.minimum(x0 + 1, wi - 1)
    wy = jnp.where(cy == 0, 1.0 - ly, ly)
    wx = jnp.where(cx == 0, 1.0 - lx, lx)
    yc = jnp.where(cy == 0, y0, y1i)
    xc = jnp.where(cx == 0, x0, x1i)
    weight = wy * wx * (0.25 * valid.astype(f32))
    idx = base + yc * wi + xc
    return idx, weight


def _coef_body(boxes_ref, bids_ref, idx_ref, w_ref):
    f32 = jnp.float32
    boxes = boxes_ref[...]
    b = bids_ref[...].astype(jnp.int32)
    x1 = boxes[:, 0:1]
    y1 = boxes[:, 1:2]
    x2 = boxes[:, 2:3]
    y2 = boxes[:, 3:4]
    area = (x2 - x1) * (y2 - y1)
    s = jnp.sqrt(jnp.maximum(area, 0.0))
    t = jnp.floor(4.0 + jnp.log2(s / 224.0 + 1e-6))
    lvl = jnp.clip(t, 2.0, 5.0) - 2.0

    def per_level(vals):
        v0, v1, v2, v3 = (jnp.asarray(v, f32) for v in vals)
        return jnp.where(lvl < 0.5, v0,
                         jnp.where(lvl < 1.5, v1,
                                   jnp.where(lvl < 2.5, v2, v3)))

    scale = per_level(_LEVEL_SCALE)
    wf = per_level(_LEVEL_W)
    off = per_level(_LEVEL_OFF)
    wi = wf.astype(jnp.int32)
    base = off.astype(jnp.int32) + b * (wi * wi)
    x1s = x1 * scale
    y1s = y1 * scale
    x2s = x2 * scale
    y2s = y2 * scale
    roi_w = jnp.maximum(x2s - x1s, 1.0)
    roi_h = jnp.maximum(y2s - y1s, 1.0)
    bw2 = (roi_w / 7.0) * 0.5
    bh2 = (roi_h / 7.0) * 0.5

    nb = boxes.shape[0]
    jb = lax.broadcasted_iota(jnp.int32, (nb, _ROWS_PER_ROI), 1)
    idx, w = _sample_idx_w(jb >> 4, jb & 15, x1s, y1s, bw2, bh2, wf, wi, base)
    idx_ref[...] = idx
    w_ref[...] = w


def _coefs(boxes, bids):
    blk = 64
    grid = _R // blk
    return pl.pallas_call(
        _coef_body,
        grid=(grid,),
        in_specs=[
            pl.BlockSpec((blk, 4), lambda i: (i, 0)),
            pl.BlockSpec((blk, 1), lambda i: (i, 0)),
        ],
        out_specs=[
            pl.BlockSpec((blk, _ROWS_PER_ROI), lambda i: (i, 0)),
            pl.BlockSpec((blk, _ROWS_PER_ROI), lambda i: (i, 0)),
        ],
        out_shape=[
            jax.ShapeDtypeStruct((_R, _ROWS_PER_ROI), jnp.int32),
            jax.ShapeDtypeStruct((_R, _ROWS_PER_ROI), jnp.float32),
        ],
    )(boxes, bids)


def _pack_body(t_ref, o_ref):
    x = t_ref[...]
    lo = lax.bitcast_convert_type(
        x[:, :128].astype(jnp.bfloat16), jnp.uint16).astype(jnp.int32)
    hi = lax.bitcast_convert_type(
        x[:, 128:].astype(jnp.bfloat16), jnp.uint16).astype(jnp.int32)
    o_ref[...] = (hi << 16) | lo


def _pack_table(table):
    rows = table.shape[0]
    blk = 1024
    return pl.pallas_call(
        _pack_body,
        grid=(pl.cdiv(rows, blk),),
        in_specs=[pl.BlockSpec((blk, _C), lambda i: (i, 0))],
        out_specs=pl.BlockSpec((blk, _C // 2), lambda i: (i, 0)),
        out_shape=jax.ShapeDtypeStruct((rows, _C // 2), jnp.int32),
    )(table)


def _lane_broadcast(vec, j):
    idx = jnp.full((16, 1), j, jnp.int32)
    return lax.gather(
        vec, idx,
        dimension_numbers=lax.GatherDimensionNumbers(
            offset_dims=(), collapsed_slice_dims=(0,), start_index_map=(0,)),
        slice_sizes=(1,),
        mode=lax.GatherScatterMode.PROMISE_IN_BOUNDS)


def _sc_pool(table, idx4, w3):
    mesh = plsc.VectorSubcoreMesh(core_axis_name="c", subcore_axis_name="s")

    @functools.partial(
        pl.kernel,
        mesh=mesh,
        out_type=jax.ShapeDtypeStruct((_R, _NCHUNK * 8, _C), jnp.float32),
        scratch_types=[
            pltpu.VMEM((_RPW, _NCHUNK, _CHUNK), jnp.int32),
            pltpu.VMEM((_RPW, _ROWS_PER_ROI), jnp.float32),
            pltpu.VMEM((_CHUNK, _C // 2), jnp.int32),
            pltpu.VMEM((_CHUNK, _C // 2), jnp.int32),
            pltpu.VMEM((_CHUNK, _C // 2), jnp.int32),
            pltpu.VMEM((8, _C), jnp.float32),
            pltpu.VMEM((8, _C), jnp.float32),
            pltpu.SemaphoreType.DMA,
            pltpu.SemaphoreType.DMA,
            pltpu.SemaphoreType.DMA,
            pltpu.SemaphoreType.DMA,
            pltpu.SemaphoreType.DMA,
        ],
    )
    def k(table_hbm, idx_hbm, w_hbm, out_hbm, idx_v, w_v, rows0, rows_a,
          rows_b, acc_a, acc_b, sem0, sem_a, sem_b, sem_wa, sem_wb):
        wid = lax.axis_index("s") * 2 + lax.axis_index("c")
        pltpu.sync_copy(idx_hbm.at[wid], idx_v)
        pltpu.sync_copy(w_hbm.at[wid], w_v)
        bufs = (rows0, rows_a, rows_b, rows_a, rows_b, rows_a, rows_b)
        sems = (sem0, sem_a, sem_b, sem_a, sem_b, sem_a, sem_b)
        accs_v = (acc_a, acc_b)
        wsems = (sem_wa, sem_wb)
        pltpu.async_copy(table_hbm.at[idx_v.at[0, 0]], rows0, sem0)

        def roi_iter(i, carry):
            r = wid * _RPW + i
            pltpu.make_async_copy(table_hbm.at[idx_v.at[i, 0]], rows0,
                                  sem0).wait()
            cps = {}
            for c in range(_NCHUNK):
                if c > 0:
                    cps[c].wait()
                if c + 1 < _NCHUNK:
                    cps[c + 1] = pltpu.async_copy(
                        table_hbm.at[idx_v.at[i, c + 1]], bufs[c + 1],
                        sems[c + 1])
                else:
                    @pl.when(i + 1 < _RPW)
                    def _():
                        pltpu.async_copy(table_hbm.at[idx_v.at[i + 1, 0]],
                                         rows0, sem0)
                buf = bufs[c]
                acc_buf = accs_v[c % 2]
                wsem = wsems[c % 2]
                wbase = c * _CHUNK
                if c >= 2:
                    pltpu.make_async_copy(
                        acc_buf, out_hbm.at[r, pl.ds(0, 8)], wsem).wait()
                else:
                    @pl.when(i > 0)
                    def _():
                        pltpu.make_async_copy(
                            acc_buf, out_hbm.at[r, pl.ds(0, 8)],
                            wsem).wait()

                def bin_iter(pw, _, c=c, buf=buf, acc_buf=acc_buf,
                             wbase=wbase):
                    accs = [jnp.zeros((16,), jnp.float32) for _ in range(16)]
                    wv = w_v[i, pl.ds(wbase + pw * _PAIRS, _PAIRS)]
                    hi_mask = jnp.full((16,), -65536, jnp.int32)
                    for j in range(_PAIRS):
                        wj = _lane_broadcast(wv, j)
                        row = pw * _PAIRS + j
                        for g in range(8):
                            u = buf[row, pl.ds(16 * g, 16)]
                            ulo = lax.bitcast_convert_type(
                                u << 16, jnp.float32)
                            uhi = lax.bitcast_convert_type(
                                u & hi_mask, jnp.float32)
                            accs[g] = accs[g] + wj * ulo
                            accs[8 + g] = accs[8 + g] + wj * uhi
                    for v in range(16):
                        acc_buf[pw, pl.ds(16 * v, 16)] = accs[v]
                    return _

                lax.fori_loop(0, _OUT, bin_iter, 0)
                pltpu.async_copy(acc_buf, out_hbm.at[r, pl.ds(c * 8, 8)],
                                 wsem)
            return carry

        lax.fori_loop(0, _RPW, roi_iter, 0)
        pltpu.make_async_copy(acc_b, out_hbm.at[0, pl.ds(0, 8)],
                              sem_wb).wait()
        pltpu.make_async_copy(acc_a, out_hbm.at[0, pl.ds(0, 8)],
                              sem_wa).wait()

    return k(table, idx4, w3)


def kernel(feat0, feat1, feat2, feat3, boxes, batch_ids):
    def _rne16(bits):
        return bits + 32767 + ((bits >> 16) & 1)

    def _packed(f):
        lo = lax.bitcast_convert_type(f[:, :128], jnp.int32)
        hi = lax.bitcast_convert_type(f[:, 128:], jnp.int32)
        lo = (_rne16(lo) >> 16) & 0xFFFF
        hi = _rne16(hi) & -65536
        return hi | lo

    packed = [lax.optimization_barrier(_packed(f))
              for f in (feat0, feat1, feat2, feat3)]
    table = jnp.concatenate(
        [p.transpose(0, 2, 3, 1).reshape(-1, _C // 2) for p in packed],
        axis=0)
    bids = batch_ids.astype(jnp.int32).reshape(_R, 1)
    idx2, w2 = _coefs(boxes.astype(jnp.float32), bids)
    idx4 = idx2.reshape(_NW, _RPW, _NCHUNK, _CHUNK)
    w3 = w2.reshape(_NW, _RPW, _ROWS_PER_ROI)
    padded = _sc_pool(table, idx4, w3)
    pooled = padded.reshape(_R, _NCHUNK, 8, _C)[:, :, :_OUT]
    pooled = pooled.reshape(_R, _BINS, _C)
    return pooled.transpose(0, 2, 1).reshape(_R, _C, _OUT, _OUT)


# speedup vs baseline: 1.2638x; 1.0010x over previous
import functools

import jax
import jax.numpy as jnp
from jax import lax
from jax.experimental import pallas as pl
from jax.experimental.pallas import tpu as pltpu
from jax.experimental.pallas import tpu_sc as plsc

_C = 256
_OUT = 7
_BINS = _OUT * _OUT
_PAIRS = 16
_ROWS_PER_ROI = _BINS * _PAIRS
_CHUNK = 112
_NCHUNK = 7
_R = 512
_NW = 32
_RPW = _R // _NW

_LEVEL_W = (200.0, 100.0, 50.0, 25.0)
_LEVEL_SCALE = (0.25, 0.125, 0.0625, 0.03125)
_LEVEL_OFF = (0.0, 80000.0, 100000.0, 105000.0)


def _sample_idx_w(bin_i, pair_i, x1s, y1s, bw2, bh2, wf, wi, base):
    f32 = jnp.float32
    binf = bin_i.astype(f32)
    phf = jnp.floor(binf / 7.0)
    pwf = binf - 7.0 * phf
    iy = (pair_i >> 1) & 1
    ix = pair_i & 1
    cc = pair_i >> 2
    cy = cc >> 1
    cx = cc & 1
    syf = 2.0 * phf + iy.astype(f32)
    sxf = 2.0 * pwf + ix.astype(f32)
    yf = y1s + (syf + 0.5) * bh2
    xf = x1s + (sxf + 0.5) * bw2
    valid = ((yf >= -1.0) & (yf <= wf)) & ((xf >= -1.0) & (xf <= wf))
    ycl = jnp.clip(yf, 0.0, wf - 1.0)
    xcl = jnp.clip(xf, 0.0, wf - 1.0)
    y0f = jnp.floor(ycl)
    x0f = jnp.floor(xcl)
    ly = ycl - y0f
    lx = xcl - x0f
    y0 = y0f.astype(jnp.int32)
    x0 = x0f.astype(jnp.int32)
    y1i = jnp.minimum(y0 + 1, wi - 1)
    x1i = jnp.minimum(x0 + 1, wi - 1)
    wy = jnp.where(cy == 0, 1.0 - ly, ly)
    wx = jnp.where(cx == 0, 1.0 - lx, lx)
    yc = jnp.where(cy == 0, y0, y1i)
    xc = jnp.where(cx == 0, x0, x1i)
    weight = wy * wx * (0.25 * valid.astype(f32))
    idx = base + yc * wi + xc
    return idx, weight


def _coef_body(boxes_ref, bids_ref, idx_ref, w_ref):
    f32 = jnp.float32
    boxes = boxes_ref[...]
    b = bids_ref[...].astype(jnp.int32)
    x1 = boxes[:, 0:1]
    y1 = boxes[:, 1:2]
    x2 = boxes[:, 2:3]
    y2 = boxes[:, 3:4]
    area = (x2 - x1) * (y2 - y1)
    s = jnp.sqrt(jnp.maximum(area, 0.0))
    t = jnp.floor(4.0 + jnp.log2(s / 224.0 + 1e-6))
    lvl = jnp.clip(t, 2.0, 5.0) - 2.0

    def per_level(vals):
        v0, v1, v2, v3 = (jnp.asarray(v, f32) for v in vals)
        return jnp.where(lvl < 0.5, v0,
                         jnp.where(lvl < 1.5, v1,
                                   jnp.where(lvl < 2.5, v2, v3)))

    scale = per_level(_LEVEL_SCALE)
    wf = per_level(_LEVEL_W)
    off = per_level(_LEVEL_OFF)
    wi = wf.astype(jnp.int32)
    base = off.astype(jnp.int32) + b * (wi * wi)
    x1s = x1 * scale
    y1s = y1 * scale
    x2s = x2 * scale
    y2s = y2 * scale
    roi_w = jnp.maximum(x2s - x1s, 1.0)
    roi_h = jnp.maximum(y2s - y1s, 1.0)
    bw2 = (roi_w / 7.0) * 0.5
    bh2 = (roi_h / 7.0) * 0.5

    nb = boxes.shape[0]
    jb = lax.broadcasted_iota(jnp.int32, (nb, _ROWS_PER_ROI), 1)
    idx, w = _sample_idx_w(jb >> 4, jb & 15, x1s, y1s, bw2, bh2, wf, wi, base)
    idx_ref[...] = idx
    w_ref[...] = w


def _coefs(boxes, bids):
    blk = 64
    grid = _R // blk
    return pl.pallas_call(
        _coef_body,
        grid=(grid,),
        in_specs=[
            pl.BlockSpec((blk, 4), lambda i: (i, 0)),
            pl.BlockSpec((blk, 1), lambda i: (i, 0)),
        ],
        out_specs=[
            pl.BlockSpec((blk, _ROWS_PER_ROI), lambda i: (i, 0)),
            pl.BlockSpec((blk, _ROWS_PER_ROI), lambda i: (i, 0)),
        ],
        out_shape=[
            jax.ShapeDtypeStruct((_R, _ROWS_PER_ROI), jnp.int32),
            jax.ShapeDtypeStruct((_R, _ROWS_PER_ROI), jnp.float32),
        ],
    )(boxes, bids)


def _pack_body(t_ref, o_ref):
    x = t_ref[...]
    lo = lax.bitcast_convert_type(
        x[:, :128].astype(jnp.bfloat16), jnp.uint16).astype(jnp.int32)
    hi = lax.bitcast_convert_type(
        x[:, 128:].astype(jnp.bfloat16), jnp.uint16).astype(jnp.int32)
    o_ref[...] = (hi << 16) | lo


def _pack_table(table):
    rows = table.shape[0]
    blk = 1024
    return pl.pallas_call(
        _pack_body,
        grid=(pl.cdiv(rows, blk),),
        in_specs=[pl.BlockSpec((blk, _C), lambda i: (i, 0))],
        out_specs=pl.BlockSpec((blk, _C // 2), lambda i: (i, 0)),
        out_shape=jax.ShapeDtypeStruct((rows, _C // 2), jnp.int32),
    )(table)


def _lane_broadcast(vec, j):
    idx = jnp.full((16, 1), j, jnp.int32)
    return lax.gather(
        vec, idx,
        dimension_numbers=lax.GatherDimensionNumbers(
            offset_dims=(), collapsed_slice_dims=(0,), start_index_map=(0,)),
        slice_sizes=(1,),
        mode=lax.GatherScatterMode.PROMISE_IN_BOUNDS)


def _sc_pool(table, idx4, w3):
    mesh = plsc.VectorSubcoreMesh(core_axis_name="c", subcore_axis_name="s")

    @functools.partial(
        pl.kernel,
        mesh=mesh,
        out_type=jax.ShapeDtypeStruct((_R, _NCHUNK * 8, _C), jnp.float32),
        scratch_types=[
            pltpu.VMEM((_RPW, _NCHUNK, _CHUNK), jnp.int32),
            pltpu.VMEM((_RPW, _ROWS_PER_ROI), jnp.float32),
            pltpu.VMEM((_CHUNK, _C // 2), jnp.int32),
            pltpu.VMEM((_CHUNK, _C // 2), jnp.int32),
            pltpu.VMEM((_CHUNK, _C // 2), jnp.int32),
            pltpu.VMEM((8, _C), jnp.float32),
            pltpu.VMEM((8, _C), jnp.float32),
            pltpu.SemaphoreType.DMA,
            pltpu.SemaphoreType.DMA,
            pltpu.SemaphoreType.DMA,
            pltpu.SemaphoreType.DMA,
            pltpu.SemaphoreType.DMA,
        ],
    )
    def k(table_hbm, idx_hbm, w_hbm, out_hbm, idx_v, w_v, rows0, rows_a,
          rows_b, acc_a, acc_b, sem0, sem_a, sem_b, sem_wa, sem_wb):
        wid = lax.axis_index("s") * 2 + lax.axis_index("c")
        pltpu.sync_copy(idx_hbm.at[wid], idx_v)
        pltpu.sync_copy(w_hbm.at[wid], w_v)
        bufs = (rows0, rows_a, rows_b, rows_a, rows_b, rows_a, rows_b)
        sems = (sem0, sem_a, sem_b, sem_a, sem_b, sem_a, sem_b)
        accs_v = (acc_a, acc_b)
        wsems = (sem_wa, sem_wb)
        pltpu.async_copy(table_hbm.at[idx_v.at[0, 0]], rows0, sem0)

        def roi_iter(i, carry):
            r = wid * _RPW + i
            pltpu.make_async_copy(table_hbm.at[idx_v.at[i, 0]], rows0,
                                  sem0).wait()
            cps = {}
            for c in range(_NCHUNK):
                if c > 0:
                    cps[c].wait()
                if c + 1 < _NCHUNK:
                    cps[c + 1] = pltpu.async_copy(
                        table_hbm.at[idx_v.at[i, c + 1]], bufs[c + 1],
                        sems[c + 1])
                else:
                    @pl.when(i + 1 < _RPW)
                    def _():
                        pltpu.async_copy(table_hbm.at[idx_v.at[i + 1, 0]],
                                         rows0, sem0)
                buf = bufs[c]
                acc_buf = accs_v[c % 2]
                wsem = wsems[c % 2]
                wbase = c * _CHUNK
                if c >= 2:
                    pltpu.make_async_copy(
                        acc_buf, out_hbm.at[r, pl.ds(0, 8)], wsem).wait()
                else:
                    @pl.when(i > 0)
                    def _():
                        pltpu.make_async_copy(
                            acc_buf, out_hbm.at[r, pl.ds(0, 8)],
                            wsem).wait()

                def bin_iter(pw, _, c=c, buf=buf, acc_buf=acc_buf,
                             wbase=wbase):
                    accs = [jnp.zeros((16,), jnp.float32) for _ in range(16)]
                    wv = w_v[i, pl.ds(wbase + pw * _PAIRS, _PAIRS)]
                    for j in range(_PAIRS):
                        wj = _lane_broadcast(wv, j)
                        row = pw * _PAIRS + j
                        for g in range(8):
                            u = buf[row, pl.ds(16 * g, 16)]
                            ulo = lax.bitcast_convert_type(
                                u << 16, jnp.float32)
                            uhi = lax.bitcast_convert_type(u, jnp.float32)
                            accs[g] = accs[g] + wj * ulo
                            accs[8 + g] = accs[8 + g] + wj * uhi
                    for v in range(16):
                        acc_buf[pw, pl.ds(16 * v, 16)] = accs[v]
                    return _

                lax.fori_loop(0, _OUT, bin_iter, 0)
                pltpu.async_copy(acc_buf, out_hbm.at[r, pl.ds(c * 8, 8)],
                                 wsem)
            return carry

        lax.fori_loop(0, _RPW, roi_iter, 0)
        pltpu.make_async_copy(acc_b, out_hbm.at[0, pl.ds(0, 8)],
                              sem_wb).wait()
        pltpu.make_async_copy(acc_a, out_hbm.at[0, pl.ds(0, 8)],
                              sem_wa).wait()

    return k(table, idx4, w3)


def kernel(feat0, feat1, feat2, feat3, boxes, batch_ids):
    def _rne16(bits):
        return bits + 32767 + ((bits >> 16) & 1)

    def _packed(f):
        lo = lax.bitcast_convert_type(f[:, :128], jnp.int32)
        hi = lax.bitcast_convert_type(f[:, 128:], jnp.int32)
        lo = (_rne16(lo) >> 16) & 0xFFFF
        hi = _rne16(hi) & -65536
        return hi | lo

    packed = [lax.optimization_barrier(_packed(f))
              for f in (feat0, feat1, feat2, feat3)]
    table = jnp.concatenate(
        [p.transpose(0, 2, 3, 1).reshape(-1, _C // 2) for p in packed],
        axis=0)
    bids = batch_ids.astype(jnp.int32).reshape(_R, 1)
    idx2, w2 = _coefs(boxes.astype(jnp.float32), bids)
    idx4 = idx2.reshape(_NW, _RPW, _NCHUNK, _CHUNK)
    w3 = w2.reshape(_NW, _RPW, _ROWS_PER_ROI)
    padded = _sc_pool(table, idx4, w3)
    pooled = padded.reshape(_R, _NCHUNK, 8, _C)[:, :, :_OUT]
    pooled = pooled.reshape(_R, _BINS, _C)
    return pooled.transpose(0, 2, 1).reshape(_R, _C, _OUT, _OUT)
